# Initial kernel scaffold; baseline (speedup 1.0000x reference)
import jax
import jax.numpy as jnp
from jax.experimental import pallas as pl


def kernel(sequence, table, gamma, beta, pe):
    raise NotImplementedError("write your pallas kernel here")


# speedup vs baseline: 3.2794x; 3.2794x over previous
import functools

import jax
import jax.numpy as jnp
from jax import lax
from jax.experimental import pallas as pl
from jax.experimental.pallas import tpu as pltpu
from jax.experimental.pallas import tpu_sc as plsc

E = 128
NC = 2
NS = 16
NW = NC * NS


def _sc_gather_body(seq_hbm, table_hbm, out_hbm, idx_v, rows_a, rows_b, sem_a, sem_b):
    t_total = seq_hbm.shape[0]
    tpw = t_total // NW
    ch = rows_a.shape[0]
    nch = tpw // ch
    wid = lax.axis_index("s") * NC + lax.axis_index("c")
    base = wid * tpw
    pltpu.sync_copy(seq_hbm.at[pl.ds(base, tpw)], idx_v)
    bufs = (rows_a, rows_b)
    sems = (sem_a, sem_b)
    pltpu.async_copy(table_hbm.at[idx_v.at[pl.ds(0, ch)]], bufs[0], sems[0])
    for k in range(nch):
        b = k % 2
        pltpu.make_async_copy(table_hbm.at[idx_v.at[pl.ds(k * ch, ch)]],
                              bufs[b], sems[b]).wait()
        if k + 1 < nch:
            nb = (k + 1) % 2
            pltpu.async_copy(table_hbm.at[idx_v.at[pl.ds((k + 1) * ch, ch)]],
                             bufs[nb], sems[nb])
        pltpu.sync_copy(bufs[b], out_hbm.at[pl.ds(base + k * ch, ch)])


def _sc_gather(seq_flat, table):
    t_total = seq_flat.shape[0]
    ch = 400
    mesh = plsc.VectorSubcoreMesh(core_axis_name="c", subcore_axis_name="s")
    fn = pl.kernel(
        _sc_gather_body,
        out_type=jax.ShapeDtypeStruct((t_total, E), jnp.float32),
        mesh=mesh,
        scratch_types=[
            pltpu.VMEM((t_total // NW,), jnp.int32),
            pltpu.VMEM((ch, E), jnp.float32),
            pltpu.VMEM((ch, E), jnp.float32),
            pltpu.SemaphoreType.DMA,
            pltpu.SemaphoreType.DMA,
        ],
    )
    return fn(seq_flat, table)


def _tc_ln_body(nseq, seq_ref, tok_ref, pe_ref, gamma_ref, beta_ref, out_ref):
    x = tok_ref[...]
    s = seq_ref[...]
    x = jnp.where(s == 0, 0.0, x)
    pe_rep = jnp.concatenate([pe_ref[...]] * nseq, axis=0)
    x = x + pe_rep
    mean = jnp.mean(x, axis=-1, keepdims=True)
    xc = x - mean
    var = jnp.mean(xc * xc, axis=-1, keepdims=True)
    y = xc * lax.rsqrt(var + 1e-12)
    out_ref[...] = y * gamma_ref[...] + beta_ref[...]


def _tc_layernorm(seq_flat, tok, pe_l, gamma, beta):
    t, e = tok.shape
    l = pe_l.shape[0]
    nseq = 8
    rt = nseq * l
    grid = (t // rt,)
    return pl.pallas_call(
        functools.partial(_tc_ln_body, nseq),
        grid=grid,
        in_specs=[
            pl.BlockSpec((rt, 1), lambda i: (i, 0)),
            pl.BlockSpec((rt, e), lambda i: (i, 0)),
            pl.BlockSpec((l, e), lambda i: (0, 0)),
            pl.BlockSpec((1, e), lambda i: (0, 0)),
            pl.BlockSpec((1, e), lambda i: (0, 0)),
        ],
        out_specs=pl.BlockSpec((rt, e), lambda i: (i, 0)),
        out_shape=jax.ShapeDtypeStruct((t, e), jnp.float32),
    )(seq_flat.reshape(t, 1), tok, pe_l, gamma.reshape(1, e), beta.reshape(1, e))


def kernel(sequence, table, gamma, beta, pe):
    b, l = sequence.shape
    seq_flat = sequence.reshape(-1).astype(jnp.int32)
    gathered = _sc_gather(seq_flat, table)
    out = _tc_layernorm(seq_flat, gathered, pe[:l], gamma, beta)
    return out.reshape(b, l, E)
